# baseline (speedup 1.0000x reference)
import functools

import jax
import jax.numpy as jnp
from jax import lax
from jax.experimental import pallas as pl
from jax.experimental.pallas import tpu as pltpu
from jax.experimental.pallas import tpu_sc as plsc

N, D, E, D_OUT = 10000, 128, 320000, 64
HD = D // 2
NC, NS = 2, 16
NW = NC * NS
EW = E // NW
K = 128
CHUNKS = 80
EWP = CHUNKS * K
NPAD = 10240
RPT = NPAD // NS

_MESH = plsc.VectorSubcoreMesh(core_axis_name="c", subcore_axis_name="s",
                               num_cores=NC, num_subcores=NS)


@functools.partial(
    pl.kernel,
    out_type=(jax.ShapeDtypeStruct((2, NC, NPAD, HD), jnp.float32),
              jax.ShapeDtypeStruct((NW, NPAD), jnp.float32)),
    mesh=_MESH,
    compiler_params=pltpu.CompilerParams(needs_layout_passes=False,
                                        use_tc_tiling_on_sc=False),
    scratch_types=[
        pltpu.VMEM((8, K), jnp.int32),
        pltpu.VMEM((2, K, HD), jnp.float32),
        pltpu.VMEM((NPAD,), jnp.float32),
        pltpu.VMEM_SHARED((NPAD, HD), jnp.float32),
        pltpu.VMEM_SHARED((NPAD, HD), jnp.float32),
        pltpu.SemaphoreType.DMA((4,)),
        pltpu.SemaphoreType.DMA((2,)),
        pltpu.SemaphoreType.DMA((2,)),
    ],
)
def _sc_aggregate(srcdst_hbm, x2_hbm, zeros_hbm, agg_out, cnt_out,
                  idx_v, rows_v, cnt_v, x_s, agg_s, isem, gsem, ssem):
    cid = lax.axis_index("c")
    sid = lax.axis_index("s")
    wid = cid * NS + sid

    zero16 = jnp.zeros((16,), jnp.float32)
    one16 = jnp.ones((16,), jnp.float32)
    base = sid * RPT

    def load_idx(j, slot):
        pltpu.async_copy(srcdst_hbm.at[wid, j],
                         idx_v.at[pl.ds(2 * slot, 2)], isem.at[slot])

    def wait_idx(j, slot):
        pltpu.make_async_copy(srcdst_hbm.at[wid, j],
                              idx_v.at[pl.ds(2 * slot, 2)],
                              isem.at[slot]).wait()

    def start_gather(slot, b):
        pltpu.async_copy(x_s.at[idx_v.at[2 * slot]], rows_v.at[b],
                         gsem.at[b])

    def wait_gather(slot, b):
        pltpu.make_async_copy(x_s.at[idx_v.at[2 * slot]], rows_v.at[b],
                              gsem.at[b]).wait()

    def start_scatter(slot, b):
        pltpu.async_copy(rows_v.at[b], agg_s.at[idx_v.at[2 * slot + 1]],
                         ssem.at[b], add=True)

    def wait_scatter(slot, b):
        pltpu.make_async_copy(rows_v.at[b], agg_s.at[idx_v.at[2 * slot + 1]],
                              ssem.at[b]).wait()

    def hist(slot):
        for c in range(K // 16):
            idx16 = idx_v[2 * slot + 1, pl.ds(c * 16, 16)]
            plsc.addupdate_scatter(cnt_v, [idx16], one16)

    def step(j, si, do_hist, *, w1=True, w2=True, i1=True, i2=True):
        b = si % 2
        ob = 1 - b
        sn = (si + 1) % 4
        sl = (si + 3) % 4
        if w1:
            wait_idx(j + 1, sn)
        if w2:
            wait_scatter(sl, ob)
        if i1:
            start_gather(sn, ob)
        if i2:
            load_idx(j + 3, sl)
        wait_gather(si, b)
        start_scatter(si, b)
        if do_hist:
            hist(si)

    def run_pass(p, do_hist):
        pltpu.sync_copy(x2_hbm.at[p, pl.ds(base, RPT)],
                        x_s.at[pl.ds(base, RPT)])
        pltpu.sync_copy(zeros_hbm, agg_s.at[pl.ds(base, RPT)])
        load_idx(0, 0)
        load_idx(1, 1)
        load_idx(2, 2)
        wait_idx(0, 0)
        plsc.subcore_barrier()
        start_gather(0, 0)

        step(0, 0, do_hist, w2=False)
        step(1, 1, do_hist)
        step(2, 2, do_hist)
        step(3, 3, do_hist)

        @pl.loop(1, 19)
        def _(g):
            for i in range(4):
                step(4 * g + i, i, do_hist)

        step(76, 0, do_hist)
        step(77, 1, do_hist, i2=False)
        step(78, 2, do_hist, i2=False)
        step(79, 3, do_hist, w1=False, i1=False, i2=False)
        wait_scatter(3, 1)

        plsc.subcore_barrier()
        pltpu.sync_copy(agg_s.at[pl.ds(base, RPT)],
                        agg_out.at[p, cid, pl.ds(base, RPT)])

    @pl.loop(0, NPAD // 16)
    def _(r):
        cnt_v[pl.ds(r * 16, 16)] = zero16

    run_pass(0, True)
    plsc.subcore_barrier()
    run_pass(1, False)
    pltpu.sync_copy(cnt_v, cnt_out.at[wid])


BN = 1000


def _tc_skip_body(x_ref, wrs_ref, b0_ref, out_ref):
    out_ref[...] = (jnp.dot(x_ref[...], wrs_ref[...],
                            preferred_element_type=jnp.float32) + b0_ref[...])


def _tc_body(sk_ref, a00_ref, a01_ref, a10_ref, a11_ref, c_ref,
             wl_lo_ref, wl_hi_ref, w1_ref, w2_ref,
             b1_ref, b2_ref, out_ref):
    cnt = jnp.sum(c_ref[...], axis=1, keepdims=True)
    r = 1.0 / jnp.maximum(cnt, 1.0)
    mean_lo = (a00_ref[0] + a01_ref[0]) * r
    mean_hi = (a10_ref[0] + a11_ref[0]) * r
    h = (jnp.dot(mean_lo, wl_lo_ref[...], preferred_element_type=jnp.float32)
         + jnp.dot(mean_hi, wl_hi_ref[...], preferred_element_type=jnp.float32)
         + sk_ref[...])
    h = jnp.maximum(
        jnp.dot(h, w1_ref[...], preferred_element_type=jnp.float32)
        + b1_ref[...], 0.0)
    out_ref[...] = (jnp.dot(h, w2_ref[...], preferred_element_type=jnp.float32)
                    + b2_ref[...])


def kernel(x, edge_idxes, Wl, bl, Wr, Wsk, bsk, W1, b1, W2, b2):
    ei = edge_idxes[0]
    pad = EWP - EW
    src = jnp.concatenate(
        [ei[0].reshape(NW, EW),
         jnp.zeros((NW, pad), jnp.int32)], axis=1).reshape(NW, CHUNKS, K)
    dst = jnp.concatenate(
        [ei[1].reshape(NW, EW),
         jnp.full((NW, pad), N, jnp.int32)], axis=1).reshape(NW, CHUNKS, K)
    srcdst = jnp.stack([src, dst], axis=2)
    xp = jnp.pad(x, ((0, NPAD - N), (0, 0)))
    x2 = jnp.stack([xp[:, :HD], xp[:, HD:]])
    zeros = jnp.zeros((RPT, HD), jnp.float32)

    wlT = Wl.T
    wl_lo = wlT[:HD]
    wl_hi = wlT[HD:]
    wrs = (Wr + Wsk).T
    w1 = W1.T
    w2 = W2.T
    b0 = (bl + bsk).reshape(1, D)
    b1r = b1.reshape(1, D)
    b2r = b2.reshape(1, D_OUT)

    row_spec = pl.BlockSpec((BN, D), lambda i: (i, 0))
    cnt_spec = pl.BlockSpec((BN, NW), lambda i: (i, 0))
    agg_spec = lambda k: pl.BlockSpec((1, BN, HD), lambda i, k=k: (k, i, 0))
    full = lambda shape: pl.BlockSpec(shape, lambda i: (0, 0))

    skip = pl.pallas_call(
        _tc_skip_body,
        grid=(N // BN,),
        in_specs=[row_spec, full((D, D)), full((1, D))],
        out_specs=row_spec,
        out_shape=jax.ShapeDtypeStruct((N, D), jnp.float32),
    )(x, wrs, b0)

    agg4, cnt32 = _sc_aggregate(srcdst, x2, zeros)
    agg4 = agg4.reshape(4, NPAD, HD)
    cnt_t = cnt32.T[:N]

    out = pl.pallas_call(
        _tc_body,
        grid=(N // BN,),
        in_specs=[
            row_spec, agg_spec(0), agg_spec(1), agg_spec(2), agg_spec(3),
            cnt_spec,
            full((HD, D)), full((HD, D)), full((D, D)),
            full((D, D_OUT)),
            full((1, D)), full((1, D_OUT)),
        ],
        out_specs=pl.BlockSpec((BN, D_OUT), lambda i: (i, 0)),
        out_shape=jax.ShapeDtypeStruct((N, D_OUT), jnp.float32),
    )(skip, agg4, agg4, agg4, agg4, cnt_t, wl_lo, wl_hi, w1, w2,
      b1r, b2r)
    return out


# speedup vs baseline: 1.0695x; 1.0695x over previous
import functools

import jax
import jax.numpy as jnp
from jax import lax
from jax.experimental import pallas as pl
from jax.experimental.pallas import tpu as pltpu
from jax.experimental.pallas import tpu_sc as plsc

N, D, E, D_OUT = 10000, 128, 320000, 64
HD = D // 2
NC, NS = 2, 16
NW = NC * NS
EW = E // NW
K = 128
CHUNKS = 80
EWP = CHUNKS * K
NPAD = 10240
RPT = NPAD // NS

_MESH = plsc.VectorSubcoreMesh(core_axis_name="c", subcore_axis_name="s",
                               num_cores=NC, num_subcores=NS)


@functools.partial(
    pl.kernel,
    out_type=(jax.ShapeDtypeStruct((2, NC, NPAD, HD), jnp.float32),
              jax.ShapeDtypeStruct((NW, NPAD), jnp.float32)),
    mesh=_MESH,
    compiler_params=pltpu.CompilerParams(needs_layout_passes=False,
                                        use_tc_tiling_on_sc=False),
    scratch_types=[
        pltpu.VMEM((8, K), jnp.int32),
        pltpu.VMEM((2, K, HD), jnp.float32),
        pltpu.VMEM((NPAD,), jnp.float32),
        pltpu.VMEM_SHARED((NPAD, HD), jnp.float32),
        pltpu.VMEM_SHARED((NPAD, HD), jnp.float32),
        pltpu.SemaphoreType.DMA((4,)),
        pltpu.SemaphoreType.DMA((2,)),
        pltpu.SemaphoreType.DMA((2,)),
    ],
)
def _sc_aggregate(srcdst_hbm, x_hbm, zeros_hbm, agg_out, cnt_out,
                  idx_v, rows_v, cnt_v, x_s, agg_s, isem, gsem, ssem):
    cid = lax.axis_index("c")
    sid = lax.axis_index("s")
    wid = cid * NS + sid

    zero16 = jnp.zeros((16,), jnp.float32)
    one16 = jnp.ones((16,), jnp.float32)
    base = sid * RPT

    def load_idx(j, slot):
        pltpu.async_copy(srcdst_hbm.at[wid, j],
                         idx_v.at[pl.ds(2 * slot, 2)], isem.at[slot])

    def wait_idx(j, slot):
        pltpu.make_async_copy(srcdst_hbm.at[wid, j],
                              idx_v.at[pl.ds(2 * slot, 2)],
                              isem.at[slot]).wait()

    def start_gather(slot, b):
        pltpu.async_copy(x_s.at[idx_v.at[2 * slot]], rows_v.at[b],
                         gsem.at[b])

    def wait_gather(slot, b):
        pltpu.make_async_copy(x_s.at[idx_v.at[2 * slot]], rows_v.at[b],
                              gsem.at[b]).wait()

    def start_scatter(slot, b):
        pltpu.async_copy(rows_v.at[b], agg_s.at[idx_v.at[2 * slot + 1]],
                         ssem.at[b], add=True)

    def wait_scatter(slot, b):
        pltpu.make_async_copy(rows_v.at[b], agg_s.at[idx_v.at[2 * slot + 1]],
                              ssem.at[b]).wait()

    def hist(slot):
        for c in range(K // 16):
            idx16 = idx_v[2 * slot + 1, pl.ds(c * 16, 16)]
            plsc.addupdate_scatter(cnt_v, [idx16], one16)

    def step(j, si, do_hist, *, w1=True, w2=True, i1=True, i2=True):
        b = si % 2
        ob = 1 - b
        sn = (si + 1) % 4
        sl = (si + 3) % 4
        if w1:
            wait_idx(j + 1, sn)
        if w2:
            wait_scatter(sl, ob)
        if i1:
            start_gather(sn, ob)
        if i2:
            load_idx(j + 3, sl)
        wait_gather(si, b)
        start_scatter(si, b)
        if do_hist:
            hist(si)

    def run_pass(p, do_hist):
        xoff = pl.multiple_of(jnp.minimum(base, N - RPT), 8)
        pltpu.sync_copy(x_hbm.at[pl.ds(xoff, RPT), pl.ds(p * HD, HD)],
                        x_s.at[pl.ds(xoff, RPT)])
        pltpu.sync_copy(zeros_hbm, agg_s.at[pl.ds(base, RPT)])
        load_idx(0, 0)
        load_idx(1, 1)
        load_idx(2, 2)
        wait_idx(0, 0)
        plsc.subcore_barrier()
        start_gather(0, 0)

        step(0, 0, do_hist, w2=False)
        step(1, 1, do_hist)
        step(2, 2, do_hist)
        step(3, 3, do_hist)

        @pl.loop(1, 19)
        def _(g):
            for i in range(4):
                step(4 * g + i, i, do_hist)

        step(76, 0, do_hist)
        step(77, 1, do_hist, i2=False)
        step(78, 2, do_hist, i2=False)
        step(79, 3, do_hist, w1=False, i1=False, i2=False)
        wait_scatter(3, 1)

        plsc.subcore_barrier()
        pltpu.sync_copy(agg_s.at[pl.ds(base, RPT)],
                        agg_out.at[p, cid, pl.ds(base, RPT)])

    @pl.loop(0, NPAD // 16)
    def _(r):
        cnt_v[pl.ds(r * 16, 16)] = zero16

    run_pass(0, True)
    plsc.subcore_barrier()
    run_pass(1, False)
    pltpu.sync_copy(cnt_v, cnt_out.at[wid])


BN = 2000


def _tc_body(x_ref, a00_ref, a01_ref, a10_ref, a11_ref, c_ref,
             wl_lo_ref, wl_hi_ref, wrs_ref, w1_ref, w2_ref,
             b0_ref, b1_ref, b2_ref, out_ref):
    cnt = jnp.sum(c_ref[...], axis=1, keepdims=True)
    r = 1.0 / jnp.maximum(cnt, 1.0)
    mean_lo = (a00_ref[0] + a01_ref[0]) * r
    mean_hi = (a10_ref[0] + a11_ref[0]) * r
    h = (jnp.dot(mean_lo, wl_lo_ref[...], preferred_element_type=jnp.float32)
         + jnp.dot(mean_hi, wl_hi_ref[...], preferred_element_type=jnp.float32)
         + jnp.dot(x_ref[...], wrs_ref[...], preferred_element_type=jnp.float32)
         + b0_ref[...])
    h = jnp.maximum(
        jnp.dot(h, w1_ref[...], preferred_element_type=jnp.float32)
        + b1_ref[...], 0.0)
    out_ref[...] = (jnp.dot(h, w2_ref[...], preferred_element_type=jnp.float32)
                    + b2_ref[...])


def kernel(x, edge_idxes, Wl, bl, Wr, Wsk, bsk, W1, b1, W2, b2):
    ei = edge_idxes[0]
    pad = EWP - EW
    src = jnp.concatenate(
        [ei[0].reshape(NW, EW),
         jnp.zeros((NW, pad), jnp.int32)], axis=1).reshape(NW, CHUNKS, K)
    dst = jnp.concatenate(
        [ei[1].reshape(NW, EW),
         jnp.full((NW, pad), N, jnp.int32)], axis=1).reshape(NW, CHUNKS, K)
    srcdst = jnp.stack([src, dst], axis=2)
    zeros = jnp.zeros((RPT, HD), jnp.float32)
    agg4, cnt32 = _sc_aggregate(srcdst, x, zeros)
    agg4 = agg4.reshape(4, NPAD, HD)
    cnt_t = cnt32.T[:N]

    wlT = Wl.T
    wl_lo = wlT[:HD]
    wl_hi = wlT[HD:]
    wrs = (Wr + Wsk).T
    w1 = W1.T
    w2 = W2.T
    b0 = (bl + bsk).reshape(1, D)
    b1r = b1.reshape(1, D)
    b2r = b2.reshape(1, D_OUT)

    row_spec = pl.BlockSpec((BN, D), lambda i: (i, 0))
    cnt_spec = pl.BlockSpec((BN, NW), lambda i: (i, 0))
    agg_spec = lambda k: pl.BlockSpec((1, BN, HD), lambda i, k=k: (k, i, 0))
    full = lambda shape: pl.BlockSpec(shape, lambda i: (0, 0))
    out = pl.pallas_call(
        _tc_body,
        grid=(N // BN,),
        in_specs=[
            row_spec, agg_spec(0), agg_spec(1), agg_spec(2), agg_spec(3),
            cnt_spec,
            full((HD, D)), full((HD, D)), full((D, D)), full((D, D)),
            full((D, D_OUT)),
            full((1, D)), full((1, D)), full((1, D_OUT)),
        ],
        out_specs=pl.BlockSpec((BN, D_OUT), lambda i: (i, 0)),
        out_shape=jax.ShapeDtypeStruct((N, D_OUT), jnp.float32),
    )(x, agg4, agg4, agg4, agg4, cnt_t, wl_lo, wl_hi, wrs, w1, w2,
      b0, b1r, b2r)
    return out


# speedup vs baseline: 1.1927x; 1.1153x over previous
import functools

import jax
import jax.numpy as jnp
from jax import lax
from jax.experimental import pallas as pl
from jax.experimental.pallas import tpu as pltpu
from jax.experimental.pallas import tpu_sc as plsc

N, D, E, D_OUT = 10000, 128, 320000, 64
HD = D // 2
NC, NS = 2, 16
NW = NC * NS
EW = E // NW
K = 128
CHUNKS = 80
EWP = CHUNKS * K
NPAD = 10240
RPT = NPAD // NS

_MESH = plsc.VectorSubcoreMesh(core_axis_name="c", subcore_axis_name="s",
                               num_cores=NC, num_subcores=NS)


@functools.partial(
    pl.kernel,
    out_type=(jax.ShapeDtypeStruct((2, NC, NPAD, HD), jnp.float32),
              jax.ShapeDtypeStruct((NW, NPAD), jnp.float32)),
    mesh=_MESH,
    compiler_params=pltpu.CompilerParams(needs_layout_passes=False,
                                        use_tc_tiling_on_sc=False),
    scratch_types=[
        pltpu.VMEM((8, K), jnp.int32),
        pltpu.VMEM((2, K, HD), jnp.float32),
        pltpu.VMEM((NPAD,), jnp.float32),
        pltpu.VMEM_SHARED((NPAD, HD), jnp.float32),
        pltpu.VMEM_SHARED((NPAD, HD), jnp.float32),
        pltpu.SemaphoreType.DMA((4,)),
        pltpu.SemaphoreType.DMA((2,)),
        pltpu.SemaphoreType.DMA((2,)),
    ],
)
def _sc_aggregate(srcdst_hbm, x_hbm, zeros_hbm, agg_out, cnt_out,
                  idx_v, rows_v, cnt_v, x_s, agg_s, isem, gsem, ssem):
    cid = lax.axis_index("c")
    sid = lax.axis_index("s")
    wid = cid * NS + sid

    zero16 = jnp.zeros((16,), jnp.float32)
    one16 = jnp.ones((16,), jnp.float32)
    base = sid * RPT

    def load_idx(j, slot):
        pltpu.async_copy(srcdst_hbm.at[wid, j],
                         idx_v.at[pl.ds(2 * slot, 2)], isem.at[slot])

    def wait_idx(j, slot):
        pltpu.make_async_copy(srcdst_hbm.at[wid, j],
                              idx_v.at[pl.ds(2 * slot, 2)],
                              isem.at[slot]).wait()

    def start_gather(slot, b):
        pltpu.async_copy(x_s.at[idx_v.at[2 * slot]], rows_v.at[b],
                         gsem.at[b])

    def wait_gather(slot, b):
        pltpu.make_async_copy(x_s.at[idx_v.at[2 * slot]], rows_v.at[b],
                              gsem.at[b]).wait()

    def start_scatter(slot, b):
        pltpu.async_copy(rows_v.at[b], agg_s.at[idx_v.at[2 * slot + 1]],
                         ssem.at[b], add=True)

    def wait_scatter(slot, b):
        pltpu.make_async_copy(rows_v.at[b], agg_s.at[idx_v.at[2 * slot + 1]],
                              ssem.at[b]).wait()

    def hist(slot):
        for c in range(K // 16):
            idx16 = idx_v[2 * slot + 1, pl.ds(c * 16, 16)]
            plsc.addupdate_scatter(cnt_v, [idx16], one16)

    def step(j, si, do_hist, *, w1=True, w2=True, i1=True, i2=True):
        b = si % 2
        ob = 1 - b
        sn = (si + 1) % 4
        sl = (si + 3) % 4
        if w1:
            wait_idx(j + 1, sn)
        if w2:
            wait_scatter(sl, ob)
        if i1:
            start_gather(sn, ob)
        if i2:
            load_idx(j + 3, sl)
        wait_gather(si, b)
        start_scatter(si, b)
        if do_hist:
            hist(si)

    def run_pass(p, do_hist):
        xoff = pl.multiple_of(jnp.minimum(base, N - RPT), 8)
        pltpu.sync_copy(x_hbm.at[pl.ds(xoff, RPT), pl.ds(p * HD, HD)],
                        x_s.at[pl.ds(xoff, RPT)])
        pltpu.sync_copy(zeros_hbm, agg_s.at[pl.ds(base, RPT)])
        load_idx(0, 0)
        load_idx(1, 1)
        load_idx(2, 2)
        wait_idx(0, 0)
        plsc.subcore_barrier()
        start_gather(0, 0)

        step(0, 0, do_hist, w2=False)
        step(1, 1, do_hist)
        step(2, 2, do_hist)
        step(3, 3, do_hist)

        @pl.loop(1, 19)
        def _(g):
            for i in range(4):
                step(4 * g + i, i, do_hist)

        step(76, 0, do_hist)
        step(77, 1, do_hist, i2=False)
        step(78, 2, do_hist, i2=False)
        step(79, 3, do_hist, w1=False, i1=False, i2=False)
        wait_scatter(3, 1)

        plsc.subcore_barrier()
        pltpu.sync_copy(agg_s.at[pl.ds(base, RPT)],
                        agg_out.at[p, cid, pl.ds(base, RPT)])

    @pl.loop(0, NPAD // 16)
    def _(r):
        cnt_v[pl.ds(r * 16, 16)] = zero16

    run_pass(0, True)
    plsc.subcore_barrier()
    run_pass(1, False)
    pltpu.sync_copy(cnt_v, cnt_out.at[wid])


BN = 2000


def _tc_body(x_ref, a00_ref, a01_ref, a10_ref, a11_ref, c_ref,
             wl_lo_ref, wl_hi_ref, wrs_ref, w1_ref, w2_ref,
             b0_ref, b1_ref, b2_ref, out_ref):
    cnt = jnp.sum(c_ref[...], axis=1, keepdims=True)
    r = 1.0 / jnp.maximum(cnt, 1.0)
    mean_lo = (a00_ref[0] + a01_ref[0]) * r
    mean_hi = (a10_ref[0] + a11_ref[0]) * r
    h = (jnp.dot(mean_lo, wl_lo_ref[...], preferred_element_type=jnp.float32)
         + jnp.dot(mean_hi, wl_hi_ref[...], preferred_element_type=jnp.float32)
         + jnp.dot(x_ref[...], wrs_ref[...], preferred_element_type=jnp.float32)
         + b0_ref[...])
    h = jnp.maximum(
        jnp.dot(h, w1_ref[...], preferred_element_type=jnp.float32)
        + b1_ref[...], 0.0)
    out_ref[...] = (jnp.dot(h, w2_ref[...], preferred_element_type=jnp.float32)
                    + b2_ref[...])


def kernel(x, edge_idxes, Wl, bl, Wr, Wsk, bsk, W1, b1, W2, b2):
    ei = edge_idxes[0]
    pad = EWP - EW
    pad_vals = jnp.broadcast_to(
        jnp.array([0, N], jnp.int32).reshape(2, 1, 1), (2, NW, pad))
    padded = jnp.concatenate([ei.reshape(2, NW, EW), pad_vals], axis=2)
    srcdst = padded.reshape(2, NW, CHUNKS, K).transpose(1, 2, 0, 3)
    zeros = jnp.zeros((RPT, HD), jnp.float32)
    agg4, cnt32 = _sc_aggregate(srcdst, x, zeros)
    agg4 = agg4.reshape(4, NPAD, HD)
    cnt_t = cnt32.T[:N]

    wlT = Wl.T
    wl_lo = wlT[:HD]
    wl_hi = wlT[HD:]
    wrs = (Wr + Wsk).T
    w1 = W1.T
    w2 = W2.T
    b0 = (bl + bsk).reshape(1, D)
    b1r = b1.reshape(1, D)
    b2r = b2.reshape(1, D_OUT)

    row_spec = pl.BlockSpec((BN, D), lambda i: (i, 0))
    cnt_spec = pl.BlockSpec((BN, NW), lambda i: (i, 0))
    agg_spec = lambda k: pl.BlockSpec((1, BN, HD), lambda i, k=k: (k, i, 0))
    full = lambda shape: pl.BlockSpec(shape, lambda i: (0, 0))
    out = pl.pallas_call(
        _tc_body,
        grid=(N // BN,),
        in_specs=[
            row_spec, agg_spec(0), agg_spec(1), agg_spec(2), agg_spec(3),
            cnt_spec,
            full((HD, D)), full((HD, D)), full((D, D)), full((D, D)),
            full((D, D_OUT)),
            full((1, D)), full((1, D)), full((1, D_OUT)),
        ],
        out_specs=pl.BlockSpec((BN, D_OUT), lambda i: (i, 0)),
        out_shape=jax.ShapeDtypeStruct((N, D_OUT), jnp.float32),
    )(x, agg4, agg4, agg4, agg4, cnt_t, wl_lo, wl_hi, wrs, w1, w2,
      b0, b1r, b2r)
    return out
